# baseline (speedup 1.0000x reference)
import jax
import jax.numpy as jnp
from jax.experimental import pallas as pl
from jax.experimental.pallas import tpu as pltpu

_BR = 2048
_SLOTS = 6


_WAYS = 2
_HR = _BR // _WAYS


def _copy_body(x_hbm, o_hbm, buf, rsem, wsem):
    rows = x_hbm.shape[0]
    chunks = rows // _BR

    def reads(i):
        s = i % _SLOTS
        return [pltpu.make_async_copy(
            x_hbm.at[pl.ds(i * _BR + w * _HR, _HR), :],
            buf.at[s, pl.ds(w * _HR, _HR)],
            rsem.at[s, w]) for w in range(_WAYS)]

    def writes(i):
        s = i % _SLOTS
        return [pltpu.make_async_copy(
            buf.at[s, pl.ds(w * _HR, _HR)],
            o_hbm.at[pl.ds(i * _BR + w * _HR, _HR), :],
            wsem.at[s, w]) for w in range(_WAYS)]

    def start(cs):
        for c in cs:
            c.start()

    def wait(cs):
        for c in cs:
            c.wait()

    for i in range(min(_SLOTS, chunks)):
        start(reads(i))
    for i in range(chunks):
        wait(reads(i))
        start(writes(i))
        if i + _SLOTS < chunks:
            wait(writes(i))
            start(reads(i + _SLOTS))
    for i in range(max(chunks - _SLOTS, 0), chunks):
        wait(writes(i))


def kernel(x, relative_position_bias_table):
    del relative_position_bias_table
    b, s, d = x.shape
    x2 = x.reshape(b * s, d)
    out = pl.pallas_call(
        _copy_body,
        in_specs=[pl.BlockSpec(memory_space=pl.ANY)],
        out_specs=pl.BlockSpec(memory_space=pl.ANY),
        out_shape=jax.ShapeDtypeStruct((b * s, d), x.dtype),
        scratch_shapes=[
            pltpu.VMEM((_SLOTS, _BR, d), x.dtype),
            pltpu.SemaphoreType.DMA((_SLOTS, _WAYS)),
            pltpu.SemaphoreType.DMA((_SLOTS, _WAYS)),
        ],
    )(x2)
    return out.reshape(b, s, d)
